# baseline (speedup 1.0000x reference)
import functools

import jax
import jax.numpy as jnp
from jax import lax
from jax.experimental import pallas as pl
from jax.experimental.pallas import tpu as pltpu
from jax.experimental.pallas import tpu_sc as plsc

_NC = 2
_NS = 16
_NW = _NC * _NS
_L = 16


_CH = 128


def _build_sc(B, Dc, Dr, Db, TW, off_r, off_b):
    BPW = B // _NW
    NCHK = BPW // _CH
    NG = _CH // _L
    mesh = plsc.VectorSubcoreMesh(core_axis_name="c", subcore_axis_name="s")

    @functools.partial(
        pl.kernel,
        mesh=mesh,
        compiler_params=pltpu.CompilerParams(
            needs_layout_passes=False, use_tc_tiling_on_sc=True,
            skip_device_barrier=True),
        out_type=(
            jax.ShapeDtypeStruct((B, Dc), jnp.float32),
            jax.ShapeDtypeStruct((B, Dr), jnp.float32),
            jax.ShapeDtypeStruct((B, Db), jnp.float32),
        ),
        scratch_types=[
            pltpu.VMEM((3 * BPW,), jnp.int32),
            pltpu.VMEM((TW,), jnp.float32),
            [pltpu.VMEM((_CH, Dc), jnp.float32) for _ in range(2)],
            [pltpu.VMEM((_CH, Dr), jnp.float32) for _ in range(2)],
            [pltpu.VMEM((_CH, Db), jnp.float32) for _ in range(2)],
            pltpu.SemaphoreType.DMA,
            [pltpu.SemaphoreType.DMA for _ in range(2)],
        ],
    )
    def sc_kernel(xi_hbm, tab_hbm,
                  out_c, out_r, out_b,
                  idx_v, tab_v, rc_v, rr_v, rb_v,
                  sem_in, sem_out):
        wid = lax.axis_index("s") * _NC + lax.axis_index("c")
        base = wid * BPW

        cps_in = [
            pltpu.async_copy(tab_hbm, tab_v, sem_in),
            pltpu.async_copy(
                xi_hbm.at[pl.ds(base, BPW)], idx_v.at[pl.ds(0, BPW)],
                sem_in),
            pltpu.async_copy(
                xi_hbm.at[pl.ds(B + base, BPW)],
                idx_v.at[pl.ds(BPW, BPW)], sem_in),
            pltpu.async_copy(
                xi_hbm.at[pl.ds(2 * B + base, BPW)],
                idx_v.at[pl.ds(2 * BPW, BPW)], sem_in),
        ]
        for cp in cps_in:
            cp.wait()

        iota = jnp.arange(_L, dtype=jnp.int32)
        pending = [None, None]

        for k in range(NCHK):
            pb = k % 2
            if pending[pb] is not None:
                for cp in pending[pb]:
                    cp.wait()
            rc, rr, rb = rc_v[pb], rr_v[pb], rb_v[pb]

            def group(g, carry, k=k, rc=rc, rr=rr, rb=rb):
                rows = g * _L + iota
                ic = idx_v[pl.ds(k * _CH + g * _L, _L)]
                ir = idx_v[pl.ds(BPW + k * _CH + g * _L, _L)]
                ib = idx_v[pl.ds(2 * BPW + k * _CH + g * _L, _L)]
                oc = ic * Dc
                orr = ir * Dr + off_r
                ob = ib * Db + off_b
                for d in range(Dc):
                    v = plsc.load_gather(tab_v, [oc + d])
                    plsc.store_scatter(
                        rc, [rows, jnp.full((_L,), d, jnp.int32)], v)
                for d in range(Dr):
                    v = plsc.load_gather(tab_v, [orr + d])
                    plsc.store_scatter(
                        rr, [rows, jnp.full((_L,), d, jnp.int32)], v)
                for d in range(Db):
                    v = plsc.load_gather(tab_v, [ob + d])
                    plsc.store_scatter(
                        rb, [rows, jnp.full((_L,), d, jnp.int32)], v)
                return carry

            for g in range(NG):
                group(g, 0)

            row_sl = pl.ds(base + k * _CH, _CH)
            pending[pb] = [
                pltpu.async_copy(rc, out_c.at[row_sl], sem_out[pb]),
                pltpu.async_copy(rr, out_r.at[row_sl], sem_out[pb]),
                pltpu.async_copy(rb, out_b.at[row_sl], sem_out[pb]),
            ]

        for cps in pending:
            if cps is not None:
                for cp in cps:
                    cp.wait()

    return sc_kernel


def _states_body(x_ref, o_ref):
    o_ref[...] = x_ref[:, 3:]


def _states_tc(x, S):
    B, F = x.shape
    blk = 2048
    return pl.pallas_call(
        _states_body,
        grid=(B // blk,),
        in_specs=[pl.BlockSpec((blk, F), lambda i: (i, 0))],
        out_specs=pl.BlockSpec((blk, S), lambda i: (i, 0)),
        out_shape=jax.ShapeDtypeStruct((B, S), jnp.float32),
    )(x)


def kernel(x, W_char, W_role, W_buff):
    B, F = x.shape
    S = F - 3
    Dc = W_char.shape[1]
    Dr = W_role.shape[1]
    Db = W_buff.shape[1]

    nc = W_char.size
    nr = W_role.size
    nb = W_buff.size
    off_r = -(-nc // 128) * 128
    off_b = off_r + -(-nr // 128) * 128
    TW = off_b + -(-nb // 128) * 128

    xi = jnp.concatenate([
        x[:, 0].astype(jnp.int32),
        x[:, 1].astype(jnp.int32),
        x[:, 2].astype(jnp.int32)])
    tab = jnp.zeros((TW,), jnp.float32)
    tab = tab.at[:nc].set(W_char.reshape(-1))
    tab = tab.at[off_r:off_r + nr].set(W_role.reshape(-1))
    tab = tab.at[off_b:off_b + nb].set(W_buff.reshape(-1))

    out_c, out_r, out_b = _build_sc(B, Dc, Dr, Db, TW, off_r, off_b)(xi, tab)
    out_s = x[:, 3:]
    return (out_c, out_r, out_b, out_s)


# speedup vs baseline: 1.0425x; 1.0425x over previous
import functools

import jax
import jax.numpy as jnp
from jax import lax
from jax.experimental import pallas as pl
from jax.experimental.pallas import tpu as pltpu
from jax.experimental.pallas import tpu_sc as plsc

_NC = 2
_NS = 16
_NW = _NC * _NS
_L = 16


_CH = 128


def _build_sc(B, Dc, Dr, Db, TW, off_r, off_b):
    BPW = B // _NW
    NCHK = BPW // _CH
    NG = _CH // _L
    mesh = plsc.VectorSubcoreMesh(core_axis_name="c", subcore_axis_name="s")

    @functools.partial(
        pl.kernel,
        mesh=mesh,
        compiler_params=pltpu.CompilerParams(
            needs_layout_passes=False, use_tc_tiling_on_sc=True,
            skip_device_barrier=True),
        out_type=(
            jax.ShapeDtypeStruct((B, Dc), jnp.float32),
            jax.ShapeDtypeStruct((B, Dr), jnp.float32),
            jax.ShapeDtypeStruct((B, Db), jnp.float32),
        ),
        scratch_types=[
            pltpu.VMEM((3 * BPW,), jnp.int32),
            pltpu.VMEM((TW,), jnp.float32),
            [pltpu.VMEM((_CH, Dc), jnp.float32) for _ in range(2)],
            [pltpu.VMEM((_CH, Dr), jnp.float32) for _ in range(2)],
            [pltpu.VMEM((_CH, Db), jnp.float32) for _ in range(2)],
            pltpu.SemaphoreType.DMA,
            [pltpu.SemaphoreType.DMA for _ in range(2)],
        ],
    )
    def sc_kernel(xi_hbm, tab_hbm,
                  out_c, out_r, out_b,
                  idx_v, tab_v, rc_v, rr_v, rb_v,
                  sem_in, sem_out):
        wid = lax.axis_index("s") * _NC + lax.axis_index("c")
        base = wid * BPW

        cps_in = [
            pltpu.async_copy(tab_hbm, tab_v, sem_in),
            pltpu.async_copy(
                xi_hbm.at[pl.ds(base, BPW)], idx_v.at[pl.ds(0, BPW)],
                sem_in),
            pltpu.async_copy(
                xi_hbm.at[pl.ds(B + base, BPW)],
                idx_v.at[pl.ds(BPW, BPW)], sem_in),
            pltpu.async_copy(
                xi_hbm.at[pl.ds(2 * B + base, BPW)],
                idx_v.at[pl.ds(2 * BPW, BPW)], sem_in),
        ]
        for cp in cps_in:
            cp.wait()

        iota = jnp.arange(_L, dtype=jnp.int32)
        pending = [None, None]

        for k in range(NCHK):
            pb = k % 2
            if pending[pb] is not None:
                for cp in pending[pb]:
                    cp.wait()
            rc, rr, rb = rc_v[pb], rr_v[pb], rb_v[pb]

            def group(g, carry, k=k, rc=rc, rr=rr, rb=rb):
                rows = g * _L + iota
                ic = idx_v[pl.ds(k * _CH + g * _L, _L)]
                ir = idx_v[pl.ds(BPW + k * _CH + g * _L, _L)]
                ib = idx_v[pl.ds(2 * BPW + k * _CH + g * _L, _L)]
                oc = ic * Dc
                orr = ir * Dr + off_r
                ob = ib * Db + off_b
                for d in range(Dc):
                    v = plsc.load_gather(tab_v, [oc + d])
                    plsc.store_scatter(
                        rc, [rows, jnp.full((_L,), d, jnp.int32)], v)
                for d in range(Dr):
                    v = plsc.load_gather(tab_v, [orr + d])
                    plsc.store_scatter(
                        rr, [rows, jnp.full((_L,), d, jnp.int32)], v)
                for d in range(Db):
                    v = plsc.load_gather(tab_v, [ob + d])
                    plsc.store_scatter(
                        rb, [rows, jnp.full((_L,), d, jnp.int32)], v)
                return carry

            lax.fori_loop(0, NG, group, 0)

            row_sl = pl.ds(base + k * _CH, _CH)
            pending[pb] = [
                pltpu.async_copy(rc, out_c.at[row_sl], sem_out[pb]),
                pltpu.async_copy(rr, out_r.at[row_sl], sem_out[pb]),
                pltpu.async_copy(rb, out_b.at[row_sl], sem_out[pb]),
            ]

        for cps in pending:
            if cps is not None:
                for cp in cps:
                    cp.wait()

    return sc_kernel


def _states_body(x_ref, o_ref):
    o_ref[...] = x_ref[:, 3:]


def _states_tc(x, S):
    B, F = x.shape
    blk = 2048
    return pl.pallas_call(
        _states_body,
        grid=(B // blk,),
        in_specs=[pl.BlockSpec((blk, F), lambda i: (i, 0))],
        out_specs=pl.BlockSpec((blk, S), lambda i: (i, 0)),
        out_shape=jax.ShapeDtypeStruct((B, S), jnp.float32),
    )(x)


def kernel(x, W_char, W_role, W_buff):
    B, F = x.shape
    S = F - 3
    Dc = W_char.shape[1]
    Dr = W_role.shape[1]
    Db = W_buff.shape[1]

    nc = W_char.size
    nr = W_role.size
    nb = W_buff.size
    off_r = -(-nc // 128) * 128
    off_b = off_r + -(-nr // 128) * 128
    TW = off_b + -(-nb // 128) * 128

    xi = jnp.concatenate([
        x[:, 0].astype(jnp.int32),
        x[:, 1].astype(jnp.int32),
        x[:, 2].astype(jnp.int32)])
    tab = jnp.zeros((TW,), jnp.float32)
    tab = tab.at[:nc].set(W_char.reshape(-1))
    tab = tab.at[off_r:off_r + nr].set(W_role.reshape(-1))
    tab = tab.at[off_b:off_b + nb].set(W_buff.reshape(-1))

    out_c, out_r, out_b = _build_sc(B, Dc, Dr, Db, TW, off_r, off_b)(xi, tab)
    out_s = x[:, 3:]
    return (out_c, out_r, out_b, out_s)
